# baseline (speedup 1.0000x reference)
import functools

import jax
import jax.numpy as jnp
from jax import lax
from jax.experimental import pallas as pl
from jax.experimental.pallas import tpu as pltpu
from jax.experimental.pallas import tpu_sc as plsc

B = 512
D = 4096
E = 64
K = 8
IB = 32
ROWS = 16


def _router_kernel(x_ref, wt_ref, gates_ref, ones_ref):
    x = x_ref[...]
    wt = wt_ref[...]
    logits = jnp.dot(x, wt, preferred_element_type=jnp.float32)
    m = jnp.max(logits, axis=-1, keepdims=True)
    ex = jnp.exp(logits - m)
    s = jnp.sum(ex, axis=-1, keepdims=True)
    gates_ref[...] = ex / s
    ones_ref[...] = jnp.ones((B, E), jnp.float32)


def _sc_mask_kernel(gates_hbm, out_hbm, in_v, out_v, sem):
    wid = lax.axis_index("s") * 2 + lax.axis_index("c")
    base = wid * ROWS
    pltpu.async_copy(gates_hbm.at[pl.ds(base, ROWS)], in_v, sem).wait()
    lanes = lax.iota(jnp.int32, 16)
    perms = [lanes ^ (1 << p) for p in range(4)]

    def allmax(v):
        for p in perms:
            v = jnp.maximum(v, v[p])
        return v

    def allmin(v):
        for p in perms:
            v = jnp.minimum(v, v[p])
        return v

    for r in range(ROWS):
        work = [in_v[r, pl.ds(16 * w, 16)] for w in range(4)]
        msk = [jnp.zeros((16,), jnp.float32) for _ in range(4)]
        gidx = [lanes + 16 * w for w in range(4)]

        def body(_, carry):
            ws = carry[:4]
            ms = carry[4:]
            mx = allmax(jnp.maximum(jnp.maximum(ws[0], ws[1]),
                                    jnp.maximum(ws[2], ws[3])))
            cand = [jnp.where(ws[w] == mx, gidx[w], E) for w in range(4)]
            first = allmin(jnp.minimum(jnp.minimum(cand[0], cand[1]),
                                       jnp.minimum(cand[2], cand[3])))
            pick = [gidx[w] == first for w in range(4)]
            new_m = [jnp.where(pick[w], 1.0, ms[w]) for w in range(4)]
            new_w = [jnp.where(pick[w], -1.0, ws[w]) for w in range(4)]
            return tuple(new_w) + tuple(new_m)

        res = lax.fori_loop(0, K, body, tuple(work) + tuple(msk))
        for w in range(4):
            out_v[r, pl.ds(16 * w, 16)] = res[4 + w]
    pltpu.async_copy(out_v, out_hbm.at[pl.ds(base, ROWS)], sem).wait()


def _combine_kernel(gates_ref, mask_ref, out_ref, maskt_s):
    i = pl.program_id(0)

    @pl.when(i == 0)
    def _prep():
        maskt_s[...] = jnp.transpose(mask_ref[...])

    mt = maskt_s[...]
    g_blk = gates_ref[pl.ds(i * IB, IB), :]
    out_ref[...] = g_blk[:, :, None] * mt[None, :, :]


def kernel(x, W):
    wt = W.T
    gates, ones = pl.pallas_call(
        _router_kernel,
        out_shape=(
            jax.ShapeDtypeStruct((B, E), jnp.float32),
            jax.ShapeDtypeStruct((B, E), jnp.float32),
        ),
    )(x, wt)

    sc_mask = functools.partial(
        pl.kernel,
        mesh=plsc.VectorSubcoreMesh(core_axis_name="c", subcore_axis_name="s"),
        out_type=jax.ShapeDtypeStruct((B, E), jnp.float32),
        scratch_types=[
            pltpu.VMEM((ROWS, E), jnp.float32),
            pltpu.VMEM((ROWS, E), jnp.float32),
            pltpu.SemaphoreType.DMA,
        ],
    )(_sc_mask_kernel)
    mask = sc_mask(gates)

    outt = pl.pallas_call(
        _combine_kernel,
        grid=(B // IB,),
        in_specs=[
            pl.BlockSpec((B, E), lambda i: (0, 0)),
            pl.BlockSpec((B, E), lambda i: (0, 0)),
        ],
        out_specs=pl.BlockSpec((IB, E, B), lambda i: (i, 0, 0)),
        out_shape=jax.ShapeDtypeStruct((B, E, B), jnp.float32),
        scratch_shapes=[pltpu.VMEM((E, B), jnp.float32)],
    )(gates, mask)
    combine = jnp.transpose(outt, (0, 2, 1))
    return (combine, mask, ones)


# speedup vs baseline: 1.4698x; 1.4698x over previous
import jax
import jax.numpy as jnp
from jax.experimental import pallas as pl
from jax.experimental.pallas import tpu as pltpu

B = 512
D = 4096
E = 64
K = 8
IB = 32


def _router_kernel(x_ref, wt_ref, gates_ref, mask_ref, maskt_ref, ones_ref):
    x = x_ref[...]
    wt = wt_ref[...]
    logits = jnp.dot(x, wt, preferred_element_type=jnp.float32)
    m = jnp.max(logits, axis=-1, keepdims=True)
    ex = jnp.exp(logits - m)
    s = jnp.sum(ex, axis=-1, keepdims=True)
    gates = ex / s

    col = jax.lax.broadcasted_iota(jnp.int32, (B, E), 1)
    work = gates
    mask = jnp.zeros((B, E), jnp.float32)
    for _ in range(K):
        mx = jnp.max(work, axis=-1, keepdims=True)
        cand = jnp.where(work == mx, col, E)
        first = jnp.min(cand, axis=-1, keepdims=True)
        pick = col == first
        mask = jnp.where(pick, 1.0, mask)
        work = jnp.where(pick, -1.0, work)

    gates_ref[...] = gates
    mask_ref[...] = mask
    maskt_ref[...] = jnp.transpose(mask)
    ones_ref[...] = jnp.ones((B, E), jnp.float32)


def _combine_kernel(gates_ref, maskt_ref, out_ref):
    i = pl.program_id(0)
    mt = maskt_ref[...]
    g_blk = gates_ref[pl.ds(i * IB, IB), :]
    out_ref[...] = g_blk[:, :, None] * mt[None, :, :]


def kernel(x, W):
    wt = W.T
    gates, mask, maskt, ones = pl.pallas_call(
        _router_kernel,
        out_shape=(
            jax.ShapeDtypeStruct((B, E), jnp.float32),
            jax.ShapeDtypeStruct((B, E), jnp.float32),
            jax.ShapeDtypeStruct((E, B), jnp.float32),
            jax.ShapeDtypeStruct((B, E), jnp.float32),
        ),
    )(x, wt)

    outt = pl.pallas_call(
        _combine_kernel,
        grid=(B // IB,),
        in_specs=[
            pl.BlockSpec((B, E), lambda i: (0, 0)),
            pl.BlockSpec((E, B), lambda i: (0, 0)),
        ],
        out_specs=pl.BlockSpec((IB, E, B), lambda i: (i, 0, 0)),
        out_shape=jax.ShapeDtypeStruct((B, E, B), jnp.float32),
    )(gates, maskt)
    combine = jnp.transpose(outt, (0, 2, 1))
    return (combine, mask, ones)


# speedup vs baseline: 1.4851x; 1.0104x over previous
import jax
import jax.numpy as jnp
from jax.experimental import pallas as pl
from jax.experimental.pallas import tpu as pltpu

B = 512
D = 4096
E = 64
K = 8
IB = 32
DC = 1024
NC = D // DC


def _fused_kernel(x_hbm, wt_ref, out_ref, mask_ref, ones_ref,
                  gates_s, maskt_s, xbuf, sem):
    i = pl.program_id(0)

    @pl.when(i == 0)
    def _router():
        def chunk_copy(c, slot):
            return pltpu.make_async_copy(
                x_hbm.at[:, pl.ds(c * DC, DC)], xbuf.at[slot], sem.at[slot])

        chunk_copy(0, 0).start()
        logits = jnp.zeros((B, E), jnp.float32)
        for c in range(NC):
            slot = c % 2
            if c + 1 < NC:
                chunk_copy(c + 1, 1 - slot).start()
            chunk_copy(c, slot).wait()
            logits = logits + jnp.dot(
                xbuf[slot], wt_ref[pl.ds(c * DC, DC), :],
                preferred_element_type=jnp.float32)

        m = jnp.max(logits, axis=-1, keepdims=True)
        ex = jnp.exp(logits - m)
        s = jnp.sum(ex, axis=-1, keepdims=True)
        gates = ex / s

        col = jax.lax.broadcasted_iota(jnp.int32, (B, E), 1)
        work = gates
        mask = jnp.zeros((B, E), jnp.float32)
        for _ in range(K):
            mx = jnp.max(work, axis=-1, keepdims=True)
            cand = jnp.where(work == mx, col, E)
            first = jnp.min(cand, axis=-1, keepdims=True)
            pick = col == first
            mask = jnp.where(pick, 1.0, mask)
            work = jnp.where(pick, -1.0, work)

        gates_s[...] = gates
        maskt_s[...] = jnp.transpose(mask)
        mask_ref[...] = mask
        ones_ref[...] = jnp.ones((B, E), jnp.float32)

    mt = maskt_s[...]
    g_blk = gates_s[pl.ds(i * IB, IB), :]
    out_ref[...] = g_blk[:, :, None] * mt[None, :, :]


def kernel(x, W):
    wt = W.T
    outt, mask, ones = pl.pallas_call(
        _fused_kernel,
        grid=(B // IB,),
        in_specs=[
            pl.BlockSpec(memory_space=pl.ANY),
            pl.BlockSpec((D, E), lambda i: (0, 0)),
        ],
        out_specs=(
            pl.BlockSpec((IB, E, B), lambda i: (i, 0, 0)),
            pl.BlockSpec((B, E), lambda i: (0, 0)),
            pl.BlockSpec((B, E), lambda i: (0, 0)),
        ),
        out_shape=(
            jax.ShapeDtypeStruct((B, E, B), jnp.float32),
            jax.ShapeDtypeStruct((B, E), jnp.float32),
            jax.ShapeDtypeStruct((B, E), jnp.float32),
        ),
        scratch_shapes=[
            pltpu.VMEM((B, E), jnp.float32),
            pltpu.VMEM((E, B), jnp.float32),
            pltpu.VMEM((2, B, DC), jnp.float32),
            pltpu.SemaphoreType.DMA((2,)),
        ],
    )(x, wt)
    combine = jnp.transpose(outt, (0, 2, 1))
    return (combine, mask, ones)
